# Initial kernel scaffold; baseline (speedup 1.0000x reference)
import jax
import jax.numpy as jnp
from jax.experimental import pallas as pl


def kernel(class_i, class_j, compat_logits):
    raise NotImplementedError("write your pallas kernel here")


# speedup vs baseline: 199.7918x; 199.7918x over previous
import functools

import jax
import jax.numpy as jnp
from jax import lax
from jax.experimental import pallas as pl
from jax.experimental.pallas import tpu as pltpu
from jax.experimental.pallas import tpu_sc as plsc

NUM_CLASSES = 32
TABLE = NUM_CLASSES * NUM_CLASSES
LANES = 16


@functools.cache
def _make_lookup(n_total: int, chunk: int):
    info = plsc.get_sparse_core_info()
    nc, ns = info.num_cores, info.num_subcores
    nw = nc * ns
    assert n_total % nw == 0
    per_worker = n_total // nw
    assert per_worker % chunk == 0 and chunk % LANES == 0
    n_chunks = per_worker // chunk

    mesh = plsc.VectorSubcoreMesh(core_axis_name="c", subcore_axis_name="s")

    @functools.partial(
        pl.kernel,
        out_type=jax.ShapeDtypeStruct((n_total,), jnp.float32),
        mesh=mesh,
        compiler_params=pltpu.CompilerParams(needs_layout_passes=False),
        scratch_types=[
            pltpu.VMEM((TABLE,), jnp.float32),
            pltpu.VMEM((TABLE,), jnp.float32),
            pltpu.VMEM((chunk,), jnp.int32),
            pltpu.VMEM((chunk,), jnp.int32),
            pltpu.VMEM((chunk,), jnp.float32),
        ],
    )
    def lookup(ci_hbm, cj_hbm, lg_hbm, out_hbm, lg_v, tab_v, i_v, j_v, o_v):
        wid = lax.axis_index("s") * nc + lax.axis_index("c")

        pltpu.sync_copy(lg_hbm, lg_v)

        def build(k, carry):
            base = k * LANES
            p = lax.iota(jnp.int32, LANES) + base
            r = p >> 5
            c = p & (NUM_CLASSES - 1)
            t = (c << 5) + r
            a = lg_v[pl.ds(base, LANES)]
            b = plsc.load_gather(lg_v, [t])
            x = (a + b) * 0.5
            tab_v[pl.ds(base, LANES)] = 1.0 / (1.0 + jnp.exp(-x))
            return carry

        lax.fori_loop(0, TABLE // LANES, build, 0)

        w_base = wid * per_worker

        def do_chunk(cidx, carry):
            base = w_base + cidx * chunk
            pltpu.sync_copy(ci_hbm.at[pl.ds(base, chunk)], i_v)
            pltpu.sync_copy(cj_hbm.at[pl.ds(base, chunk)], j_v)

            def gath(k, inner):
                off = k * LANES
                iv = i_v[pl.ds(off, LANES)]
                jv = j_v[pl.ds(off, LANES)]
                idx = iv * NUM_CLASSES + jv
                o_v[pl.ds(off, LANES)] = plsc.load_gather(tab_v, [idx])
                return inner

            lax.fori_loop(0, chunk // LANES, gath, 0)
            pltpu.sync_copy(o_v, out_hbm.at[pl.ds(base, chunk)])
            return carry

        lax.fori_loop(0, n_chunks, do_chunk, 0)

    return lookup


def kernel(class_i, class_j, compat_logits):
    b, h = class_i.shape
    n = b * h
    ci = class_i.reshape(n).astype(jnp.int32)
    cj = class_j.reshape(n).astype(jnp.int32)
    lg = compat_logits.reshape(TABLE).astype(jnp.float32)
    out = _make_lookup(n, 25600)(ci, cj, lg)
    return out.reshape(b, h)


# speedup vs baseline: 237.8018x; 1.1902x over previous
import functools

import jax
import jax.numpy as jnp
from jax import lax
from jax.experimental import pallas as pl
from jax.experimental.pallas import tpu as pltpu
from jax.experimental.pallas import tpu_sc as plsc

NUM_CLASSES = 32
TABLE = NUM_CLASSES * NUM_CLASSES
LANES = 16


@functools.cache
def _make_lookup(n_total: int, chunk: int, unroll: int):
    info = plsc.get_sparse_core_info()
    nc, ns = info.num_cores, info.num_subcores
    nw = nc * ns
    assert n_total % nw == 0
    per_worker = n_total // nw
    assert per_worker % chunk == 0 and chunk % LANES == 0
    n_chunks = per_worker // chunk
    assert n_chunks >= 2

    mesh = plsc.VectorSubcoreMesh(core_axis_name="c", subcore_axis_name="s")

    @functools.partial(
        pl.kernel,
        out_type=jax.ShapeDtypeStruct((n_total,), jnp.float32),
        mesh=mesh,
        compiler_params=pltpu.CompilerParams(needs_layout_passes=False),
        scratch_types=[
            pltpu.VMEM((TABLE,), jnp.float32),
            pltpu.VMEM((TABLE,), jnp.float32),
            pltpu.VMEM((chunk,), jnp.int32),
            pltpu.VMEM((chunk,), jnp.int32),
            pltpu.VMEM((chunk,), jnp.int32),
            pltpu.VMEM((chunk,), jnp.int32),
            pltpu.VMEM((chunk,), jnp.float32),
            pltpu.VMEM((chunk,), jnp.float32),
            pltpu.SemaphoreType.DMA,
            pltpu.SemaphoreType.DMA,
            pltpu.SemaphoreType.DMA,
            pltpu.SemaphoreType.DMA,
            pltpu.SemaphoreType.DMA,
            pltpu.SemaphoreType.DMA,
        ],
    )
    def lookup(ci_hbm, cj_hbm, lg_hbm, out_hbm,
               lg_v, tab_v, i0, i1, j0, j1, o0, o1,
               si0, si1, sj0, sj1, so0, so1):
        wid = lax.axis_index("s") * nc + lax.axis_index("c")
        w_base = wid * per_worker
        ibufs, jbufs, obufs = (i0, i1), (j0, j1), (o0, o1)
        isems, jsems, osems = (si0, si1), (sj0, sj1), (so0, so1)

        def start_in(c):
            s = c % 2
            base = w_base + c * chunk
            di = pltpu.async_copy(ci_hbm.at[pl.ds(base, chunk)], ibufs[s], isems[s])
            dj = pltpu.async_copy(cj_hbm.at[pl.ds(base, chunk)], jbufs[s], jsems[s])
            return di, dj

        in_descs = {0: start_in(0), 1: start_in(1)}

        pltpu.sync_copy(lg_hbm, lg_v)

        @plsc.parallel_loop(0, TABLE, LANES)
        def build(base):
            p = lax.iota(jnp.int32, LANES) + base
            r = p >> 5
            c = p & (NUM_CLASSES - 1)
            t = (c << 5) + r
            a = lg_v[pl.ds(base, LANES)]
            b = plsc.load_gather(lg_v, [t])
            x = (a + b) * 0.5
            tab_v[pl.ds(base, LANES)] = 1.0 / (1.0 + jnp.exp(-x))

        out_descs = {}
        for c in range(n_chunks):
            s = c % 2
            di, dj = in_descs[c]
            di.wait()
            dj.wait()
            if c >= 2:
                out_descs[c - 2].wait()
            ib, jb, ob = ibufs[s], jbufs[s], obufs[s]

            def gath(off, ib=ib, jb=jb, ob=ob):
                iv = ib[pl.ds(off, LANES)]
                jv = jb[pl.ds(off, LANES)]
                idx = iv * NUM_CLASSES + jv
                ob[pl.ds(off, LANES)] = plsc.load_gather(tab_v, [idx])

            plsc.parallel_loop(0, chunk, LANES, unroll=unroll)(gath)

            base = w_base + c * chunk
            out_descs[c] = pltpu.async_copy(
                ob, out_hbm.at[pl.ds(base, chunk)], osems[s])
            if c + 2 < n_chunks:
                in_descs[c + 2] = start_in(c + 2)
        out_descs[n_chunks - 2].wait()
        out_descs[n_chunks - 1].wait()

    return lookup


def kernel(class_i, class_j, compat_logits):
    b, h = class_i.shape
    n = b * h
    ci = class_i.reshape(n).astype(jnp.int32)
    cj = class_j.reshape(n).astype(jnp.int32)
    lg = compat_logits.reshape(TABLE).astype(jnp.float32)
    out = _make_lookup(n, 12800, 8)(ci, cj, lg)
    return out.reshape(b, h)


# speedup vs baseline: 398.9392x; 1.6776x over previous
import functools

import jax
import jax.numpy as jnp
from jax import lax
from jax.experimental import pallas as pl
from jax.experimental.pallas import tpu as pltpu
from jax.experimental.pallas import tpu_sc as plsc

NUM_CLASSES = 32
TABLE = NUM_CLASSES * NUM_CLASSES
LANES = 16


@functools.cache
def _make_lookup(n_rows: int, n_cols: int, rows_per_chunk: int, unroll: int):
    info = plsc.get_sparse_core_info()
    nc, ns = info.num_cores, info.num_subcores
    nw = nc * ns
    assert n_rows % nw == 0
    rows_per_worker = n_rows // nw
    assert rows_per_worker % rows_per_chunk == 0
    n_chunks = rows_per_worker // rows_per_chunk
    assert n_chunks >= 2
    col_offs = list(range(0, n_cols - LANES + 1, LANES))
    if col_offs[-1] != n_cols - LANES:
        col_offs.append(n_cols - LANES)

    mesh = plsc.VectorSubcoreMesh(core_axis_name="c", subcore_axis_name="s")

    @functools.partial(
        pl.kernel,
        out_type=jax.ShapeDtypeStruct((n_rows, n_cols), jnp.float32),
        mesh=mesh,
        compiler_params=pltpu.CompilerParams(needs_layout_passes=False),
        scratch_types=[
            pltpu.VMEM((NUM_CLASSES, NUM_CLASSES), jnp.float32),
            pltpu.VMEM((TABLE,), jnp.float32),
            pltpu.VMEM((rows_per_chunk, n_cols), jnp.int32),
            pltpu.VMEM((rows_per_chunk, n_cols), jnp.int32),
            pltpu.VMEM((rows_per_chunk, n_cols), jnp.int32),
            pltpu.VMEM((rows_per_chunk, n_cols), jnp.int32),
            pltpu.VMEM((rows_per_chunk, n_cols), jnp.float32),
            pltpu.VMEM((rows_per_chunk, n_cols), jnp.float32),
            pltpu.SemaphoreType.DMA,
            pltpu.SemaphoreType.DMA,
            pltpu.SemaphoreType.DMA,
            pltpu.SemaphoreType.DMA,
            pltpu.SemaphoreType.DMA,
            pltpu.SemaphoreType.DMA,
        ],
    )
    def lookup(ci_hbm, cj_hbm, lg_hbm, out_hbm,
               lg_v, tab_v, i0, i1, j0, j1, o0, o1,
               si0, si1, sj0, sj1, so0, so1):
        wid = lax.axis_index("s") * nc + lax.axis_index("c")
        w_row = wid * rows_per_worker
        ibufs, jbufs, obufs = (i0, i1), (j0, j1), (o0, o1)
        isems, jsems, osems = (si0, si1), (sj0, sj1), (so0, so1)

        def start_in(c):
            s = c % 2
            row = w_row + c * rows_per_chunk
            sl = pl.ds(row, rows_per_chunk)
            di = pltpu.async_copy(ci_hbm.at[sl], ibufs[s], isems[s])
            dj = pltpu.async_copy(cj_hbm.at[sl], jbufs[s], jsems[s])
            return di, dj

        in_descs = {0: start_in(0), 1: start_in(1)}

        pltpu.sync_copy(lg_hbm, lg_v)

        @plsc.parallel_loop(0, TABLE, LANES)
        def build(base):
            p = lax.iota(jnp.int32, LANES) + base
            r = p >> 5
            c = p & (NUM_CLASSES - 1)
            a = plsc.load_gather(lg_v, [r, c])
            b = plsc.load_gather(lg_v, [c, r])
            x = (a + b) * 0.5
            tab_v[pl.ds(base, LANES)] = 1.0 / (1.0 + jnp.exp(-x))

        out_descs = {}
        for c in range(n_chunks):
            s = c % 2
            di, dj = in_descs[c]
            di.wait()
            dj.wait()
            if c >= 2:
                out_descs[c - 2].wait()
            ib, jb, ob = ibufs[s], jbufs[s], obufs[s]

            def gath(r, ib=ib, jb=jb, ob=ob):
                for off in col_offs:
                    iv = ib[r, pl.ds(off, LANES)]
                    jv = jb[r, pl.ds(off, LANES)]
                    idx = iv * NUM_CLASSES + jv
                    ob[r, pl.ds(off, LANES)] = plsc.load_gather(tab_v, [idx])

            plsc.parallel_loop(0, rows_per_chunk, 1, unroll=unroll)(gath)

            row = w_row + c * rows_per_chunk
            out_descs[c] = pltpu.async_copy(
                ob, out_hbm.at[pl.ds(row, rows_per_chunk)], osems[s])
            if c + 2 < n_chunks:
                in_descs[c + 2] = start_in(c + 2)
        out_descs[n_chunks - 2].wait()
        out_descs[n_chunks - 1].wait()

    return lookup


def kernel(class_i, class_j, compat_logits):
    n_rows, n_cols = class_i.shape
    ci = class_i.astype(jnp.int32)
    cj = class_j.astype(jnp.int32)
    lg = compat_logits.astype(jnp.float32)
    return _make_lookup(n_rows, n_cols, 64, 2)(ci, cj, lg)


# speedup vs baseline: 399.0520x; 1.0003x over previous
import functools

import jax
import jax.numpy as jnp
from jax import lax
from jax.experimental import pallas as pl
from jax.experimental.pallas import tpu as pltpu
from jax.experimental.pallas import tpu_sc as plsc

NUM_CLASSES = 32
TABLE = NUM_CLASSES * NUM_CLASSES
LANES = 16


@functools.cache
def _make_lookup(n_rows: int, n_cols: int, rows_per_chunk: int, unroll: int):
    info = plsc.get_sparse_core_info()
    nc, ns = info.num_cores, info.num_subcores
    nw = nc * ns
    assert n_rows % nw == 0
    rows_per_worker = n_rows // nw
    assert rows_per_worker % rows_per_chunk == 0
    n_chunks = rows_per_worker // rows_per_chunk
    assert n_chunks >= 2
    col_offs = list(range(0, n_cols - LANES + 1, LANES))
    if col_offs[-1] != n_cols - LANES:
        col_offs.append(n_cols - LANES)

    mesh = plsc.VectorSubcoreMesh(core_axis_name="c", subcore_axis_name="s")

    @functools.partial(
        pl.kernel,
        out_type=jax.ShapeDtypeStruct((n_rows, n_cols), jnp.float32),
        mesh=mesh,
        compiler_params=pltpu.CompilerParams(
            needs_layout_passes=False, use_tc_tiling_on_sc=True),
        scratch_types=[
            pltpu.VMEM((NUM_CLASSES, NUM_CLASSES), jnp.float32),
            pltpu.VMEM((TABLE,), jnp.float32),
            pltpu.VMEM((rows_per_chunk, n_cols), jnp.int32),
            pltpu.VMEM((rows_per_chunk, n_cols), jnp.int32),
            pltpu.VMEM((rows_per_chunk, n_cols), jnp.int32),
            pltpu.VMEM((rows_per_chunk, n_cols), jnp.int32),
            pltpu.VMEM((rows_per_chunk, n_cols), jnp.float32),
            pltpu.VMEM((rows_per_chunk, n_cols), jnp.float32),
            pltpu.SemaphoreType.DMA,
            pltpu.SemaphoreType.DMA,
            pltpu.SemaphoreType.DMA,
            pltpu.SemaphoreType.DMA,
            pltpu.SemaphoreType.DMA,
            pltpu.SemaphoreType.DMA,
        ],
    )
    def lookup(ci_hbm, cj_hbm, lg_hbm, out_hbm,
               lg_v, tab_v, i0, i1, j0, j1, o0, o1,
               si0, si1, sj0, sj1, so0, so1):
        wid = lax.axis_index("s") * nc + lax.axis_index("c")
        w_row = wid * rows_per_worker
        ibufs, jbufs, obufs = (i0, i1), (j0, j1), (o0, o1)
        isems, jsems, osems = (si0, si1), (sj0, sj1), (so0, so1)

        def start_in(c):
            s = c % 2
            row = w_row + c * rows_per_chunk
            sl = pl.ds(row, rows_per_chunk)
            di = pltpu.async_copy(ci_hbm.at[sl], ibufs[s], isems[s])
            dj = pltpu.async_copy(cj_hbm.at[sl], jbufs[s], jsems[s])
            return di, dj

        in_descs = {0: start_in(0), 1: start_in(1)}

        pltpu.sync_copy(lg_hbm, lg_v)

        @plsc.parallel_loop(0, TABLE, LANES)
        def build(base):
            p = lax.iota(jnp.int32, LANES) + base
            r = p >> 5
            c = p & (NUM_CLASSES - 1)
            a = plsc.load_gather(lg_v, [r, c])
            b = plsc.load_gather(lg_v, [c, r])
            x = (a + b) * 0.5
            tab_v[pl.ds(base, LANES)] = 1.0 / (1.0 + jnp.exp(-x))

        out_descs = {}
        for c in range(n_chunks):
            s = c % 2
            di, dj = in_descs[c]
            di.wait()
            dj.wait()
            if c >= 2:
                out_descs[c - 2].wait()
            ib, jb, ob = ibufs[s], jbufs[s], obufs[s]

            def gath(r, ib=ib, jb=jb, ob=ob):
                for off in col_offs:
                    iv = ib[r, pl.ds(off, LANES)]
                    jv = jb[r, pl.ds(off, LANES)]
                    idx = iv * NUM_CLASSES + jv
                    ob[r, pl.ds(off, LANES)] = plsc.load_gather(tab_v, [idx])

            plsc.parallel_loop(0, rows_per_chunk, 1, unroll=unroll)(gath)

            row = w_row + c * rows_per_chunk
            out_descs[c] = pltpu.async_copy(
                ob, out_hbm.at[pl.ds(row, rows_per_chunk)], osems[s])
            if c + 2 < n_chunks:
                in_descs[c + 2] = start_in(c + 2)
        out_descs[n_chunks - 2].wait()
        out_descs[n_chunks - 1].wait()

    return lookup


def kernel(class_i, class_j, compat_logits):
    n_rows, n_cols = class_i.shape
    ci = class_i.astype(jnp.int32)
    cj = class_j.astype(jnp.int32)
    lg = compat_logits.astype(jnp.float32)
    return _make_lookup(n_rows, n_cols, 64, 2)(ci, cj, lg)


# speedup vs baseline: 601.6351x; 1.5077x over previous
import functools

import jax
import jax.numpy as jnp
from jax import lax
from jax.experimental import pallas as pl
from jax.experimental.pallas import tpu as pltpu
from jax.experimental.pallas import tpu_sc as plsc

NUM_CLASSES = 32
TABLE = NUM_CLASSES * NUM_CLASSES
LANES = 16


@functools.cache
def _make_lookup(n_rows: int, n_cols: int, rows_per_chunk: int, unroll: int):
    info = plsc.get_sparse_core_info()
    nc, ns = info.num_cores, info.num_subcores
    nw = nc * ns
    assert n_cols % (nw * 128) == 0
    cols_per_worker = n_cols // nw
    assert n_rows % rows_per_chunk == 0 and rows_per_chunk % 8 == 0
    n_chunks = n_rows // rows_per_chunk
    assert n_chunks >= 2

    mesh = plsc.VectorSubcoreMesh(core_axis_name="c", subcore_axis_name="s")

    @functools.partial(
        pl.kernel,
        out_type=jax.ShapeDtypeStruct((n_rows, n_cols), jnp.float32),
        mesh=mesh,
        compiler_params=pltpu.CompilerParams(
            needs_layout_passes=False, use_tc_tiling_on_sc=True),
        scratch_types=[
            pltpu.VMEM((NUM_CLASSES, NUM_CLASSES), jnp.float32),
            pltpu.VMEM((TABLE,), jnp.float32),
            pltpu.VMEM((rows_per_chunk, cols_per_worker), jnp.int32),
            pltpu.VMEM((rows_per_chunk, cols_per_worker), jnp.int32),
            pltpu.VMEM((rows_per_chunk, cols_per_worker), jnp.int32),
            pltpu.VMEM((rows_per_chunk, cols_per_worker), jnp.int32),
            pltpu.VMEM((rows_per_chunk, cols_per_worker), jnp.float32),
            pltpu.VMEM((rows_per_chunk, cols_per_worker), jnp.float32),
            pltpu.SemaphoreType.DMA,
            pltpu.SemaphoreType.DMA,
            pltpu.SemaphoreType.DMA,
            pltpu.SemaphoreType.DMA,
            pltpu.SemaphoreType.DMA,
            pltpu.SemaphoreType.DMA,
        ],
    )
    def lookup(ci_hbm, cj_hbm, lg_hbm, out_hbm,
               lg_v, tab_v, i0, i1, j0, j1, o0, o1,
               si0, si1, sj0, sj1, so0, so1):
        wid = lax.axis_index("s") * nc + lax.axis_index("c")
        w_col = wid * cols_per_worker
        ibufs, jbufs, obufs = (i0, i1), (j0, j1), (o0, o1)
        isems, jsems, osems = (si0, si1), (sj0, sj1), (so0, so1)

        def start_in(c):
            s = c % 2
            sl = (pl.ds(c * rows_per_chunk, rows_per_chunk),
                  pl.ds(w_col, cols_per_worker))
            di = pltpu.async_copy(ci_hbm.at[sl], ibufs[s], isems[s])
            dj = pltpu.async_copy(cj_hbm.at[sl], jbufs[s], jsems[s])
            return di, dj

        in_descs = {0: start_in(0), 1: start_in(1)}

        pltpu.sync_copy(lg_hbm, lg_v)

        @plsc.parallel_loop(0, TABLE, LANES)
        def build(base):
            p = lax.iota(jnp.int32, LANES) + base
            r = p >> 5
            c = p & (NUM_CLASSES - 1)
            a = plsc.load_gather(lg_v, [r, c])
            b = plsc.load_gather(lg_v, [c, r])
            x = (a + b) * 0.5
            tab_v[pl.ds(base, LANES)] = 1.0 / (1.0 + jnp.exp(-x))

        col_offs = list(range(0, cols_per_worker, LANES))
        out_descs = {}
        for c in range(n_chunks):
            s = c % 2
            di, dj = in_descs[c]
            di.wait()
            dj.wait()
            if c >= 2:
                out_descs[c - 2].wait()
            ib, jb, ob = ibufs[s], jbufs[s], obufs[s]

            def gath(r, ib=ib, jb=jb, ob=ob):
                for off in col_offs:
                    iv = ib[r, pl.ds(off, LANES)]
                    jv = jb[r, pl.ds(off, LANES)]
                    idx = iv * NUM_CLASSES + jv
                    ob[r, pl.ds(off, LANES)] = plsc.load_gather(tab_v, [idx])

            plsc.parallel_loop(0, rows_per_chunk, 1, unroll=unroll)(gath)

            out_descs[c] = pltpu.async_copy(
                ob,
                out_hbm.at[pl.ds(c * rows_per_chunk, rows_per_chunk),
                           pl.ds(w_col, cols_per_worker)],
                osems[s])
            if c + 2 < n_chunks:
                in_descs[c + 2] = start_in(c + 2)
        out_descs[n_chunks - 2].wait()
        out_descs[n_chunks - 1].wait()

    return lookup


def kernel(class_i, class_j, compat_logits):
    n_rows, n_cols = class_i.shape
    ci = class_i.astype(jnp.int32).T
    cj = class_j.astype(jnp.int32).T
    lg = compat_logits.astype(jnp.float32)
    out_t = _make_lookup(n_cols, n_rows, 40, 2)(ci, cj, lg)
    return out_t.T


# speedup vs baseline: 856.6078x; 1.4238x over previous
import functools

import jax
import jax.numpy as jnp
from jax import lax
from jax.experimental import pallas as pl
from jax.experimental.pallas import tpu as pltpu
from jax.experimental.pallas import tpu_sc as plsc

NUM_CLASSES = 32
TABLE = NUM_CLASSES * NUM_CLASSES
LANES = 16


@functools.cache
def _make_lookup(n_rows: int, n_cols: int, rows_per_chunk: int, unroll: int):
    info = plsc.get_sparse_core_info()
    nc, ns = info.num_cores, info.num_subcores
    nw = nc * ns
    assert n_cols % (nw * 128) == 0
    cols_per_worker = n_cols // nw
    assert n_rows % rows_per_chunk == 0 and rows_per_chunk % 8 == 0
    n_chunks = n_rows // rows_per_chunk
    assert n_chunks >= 2

    mesh = plsc.VectorSubcoreMesh(core_axis_name="c", subcore_axis_name="s")

    @functools.partial(
        pl.kernel,
        out_type=jax.ShapeDtypeStruct((n_rows, n_cols), jnp.float32),
        mesh=mesh,
        compiler_params=pltpu.CompilerParams(
            needs_layout_passes=False, use_tc_tiling_on_sc=True),
        scratch_types=[
            pltpu.VMEM((NUM_CLASSES, NUM_CLASSES), jnp.float32),
            pltpu.VMEM((TABLE,), jnp.float32),
            pltpu.VMEM((rows_per_chunk, cols_per_worker), jnp.int32),
            pltpu.VMEM((rows_per_chunk, cols_per_worker), jnp.int32),
            pltpu.VMEM((rows_per_chunk, cols_per_worker), jnp.int32),
            pltpu.VMEM((rows_per_chunk, cols_per_worker), jnp.int32),
            pltpu.VMEM((rows_per_chunk, cols_per_worker), jnp.float32),
            pltpu.VMEM((rows_per_chunk, cols_per_worker), jnp.float32),
            pltpu.SemaphoreType.DMA,
            pltpu.SemaphoreType.DMA,
            pltpu.SemaphoreType.DMA,
            pltpu.SemaphoreType.DMA,
            pltpu.SemaphoreType.DMA,
            pltpu.SemaphoreType.DMA,
        ],
    )
    def lookup(ci_hbm, cj_hbm, lg_hbm, out_hbm,
               lg_v, tab_v, i0, i1, j0, j1, o0, o1,
               si0, si1, sj0, sj1, so0, so1):
        wid = lax.axis_index("s") * nc + lax.axis_index("c")
        w_col = wid * cols_per_worker
        ibufs, jbufs, obufs = (i0, i1), (j0, j1), (o0, o1)
        isems, jsems, osems = (si0, si1), (sj0, sj1), (so0, so1)

        def start_in(c):
            s = c % 2
            sl = (pl.ds(c * rows_per_chunk, rows_per_chunk),
                  pl.ds(w_col, cols_per_worker))
            di = pltpu.async_copy(ci_hbm.at[sl], ibufs[s], isems[s])
            dj = pltpu.async_copy(cj_hbm.at[sl], jbufs[s], jsems[s])
            return di, dj

        in_descs = {0: start_in(0), 1: start_in(1)}

        pltpu.sync_copy(lg_hbm, lg_v)

        @plsc.parallel_loop(0, TABLE, LANES)
        def build(base):
            p = lax.iota(jnp.int32, LANES) + base
            r = p >> 5
            c = p & (NUM_CLASSES - 1)
            a = plsc.load_gather(lg_v, [r, c])
            b = plsc.load_gather(lg_v, [c, r])
            x = (a + b) * 0.5
            tab_v[pl.ds(base, LANES)] = 1.0 / (1.0 + jnp.exp(-x))

        vecs_per_row = cols_per_worker // LANES
        n_vecs = rows_per_chunk * vecs_per_row
        out_descs = {}
        for c in range(n_chunks):
            s = c % 2
            di, dj = in_descs[c]
            di.wait()
            dj.wait()
            if c >= 2:
                out_descs[c - 2].wait()
            ib, jb, ob = ibufs[s], jbufs[s], obufs[s]

            def gath(k, ib=ib, jb=jb, ob=ob):
                r = k // vecs_per_row
                off = (k % vecs_per_row) * LANES
                iv = ib[r, pl.ds(off, LANES)]
                jv = jb[r, pl.ds(off, LANES)]
                idx = iv * NUM_CLASSES + jv
                ob[r, pl.ds(off, LANES)] = plsc.load_gather(tab_v, [idx])

            plsc.parallel_loop(0, n_vecs, 1, unroll=unroll)(gath)

            out_descs[c] = pltpu.async_copy(
                ob,
                out_hbm.at[pl.ds(c * rows_per_chunk, rows_per_chunk),
                           pl.ds(w_col, cols_per_worker)],
                osems[s])
            if c + 2 < n_chunks:
                in_descs[c + 2] = start_in(c + 2)
        out_descs[n_chunks - 2].wait()
        out_descs[n_chunks - 1].wait()

    return lookup


def kernel(class_i, class_j, compat_logits):
    n_rows, n_cols = class_i.shape
    ci = class_i.astype(jnp.int32).T
    cj = class_j.astype(jnp.int32).T
    lg = compat_logits.astype(jnp.float32)
    out_t = _make_lookup(n_cols, n_rows, 40, 4)(ci, cj, lg)
    return out_t.T
